# Initial kernel scaffold; baseline (speedup 1.0000x reference)
import jax
import jax.numpy as jnp
from jax.experimental import pallas as pl


def kernel(x, params):
    raise NotImplementedError("write your pallas kernel here")


# speedup vs baseline: 1.7494x; 1.7494x over previous
import functools

import jax
import jax.numpy as jnp
from jax.experimental import pallas as pl
from jax.experimental.pallas import tpu as pltpu

_GROUP_SLICES = [(0, 9), (9, 14), (14, 18), (18, 24), (24, 29)]
_NUM_EXPERTS = 8
_TOP_K = 3
_D_IN = 29
_D_OUT = 32
_BATCH = 16384
_TILE = 2048
_INV_SQRT2 = 0.7071067811865476


def _gelu(v):
    return 0.5 * v * (1.0 + jax.lax.erf(v * _INV_SQRT2))


def _ln(z, g, b):
    mu = jnp.mean(z, axis=-1, keepdims=True)
    c = z - mu
    var = jnp.mean(c * c, axis=-1, keepdims=True)
    return g * c / jnp.sqrt(var + 1e-5) + b


def _dot(a, b):
    return jax.lax.dot_general(a, b, (((1,), (0,)), ((), ())),
                               preferred_element_type=jnp.float32,
                               precision=jax.lax.Precision.DEFAULT)


def _moe_kernel(x_ref, *refs):
    n_grid = _BATCH // _TILE
    expert_refs = refs[:32]
    gw1, gb1, gw2, gb2 = refs[32:36]
    cw1, cvec, cw2, cb2 = refs[36:40]
    out_ref, aux_ref = refs[40:42]
    freq_acc, prob_acc = refs[42:44]

    i = pl.program_id(0)
    x = x_ref[:]

    outs = []
    for e in range(_NUM_EXPERTS):
        w1 = expert_refs[4 * e][:]
        v1 = expert_refs[4 * e + 1][:]
        w2 = expert_refs[4 * e + 2][:]
        v2 = expert_refs[4 * e + 3][:]
        h = _dot(x, w1) + v1[0:1, :]
        h = _gelu(_ln(h, v1[1:2, :], v1[2:3, :]))
        o = _dot(h, w2) + v2[0:1, :]
        o = _gelu(_ln(o, v2[1:2, :], v2[2:3, :]))
        outs.append(o)

    g = _gelu(_dot(x, gw1[:]) + gb1[:])
    logits = _dot(g, gw2[:]) + gb2[:]

    iota = jax.lax.broadcasted_iota(jnp.int32, (_TILE, _NUM_EXPERTS), 1)
    work = logits
    onehots = []
    vals = []
    for _ in range(_TOP_K):
        m = jnp.max(work, axis=1, keepdims=True)
        eq = work == m
        first = jnp.min(jnp.where(eq, iota, _NUM_EXPERTS), axis=1,
                        keepdims=True)
        oh = iota == first
        onehots.append(oh)
        vals.append(m)
        work = jnp.where(oh, -jnp.inf, work)
    e1 = jnp.exp(vals[1] - vals[0])
    e2 = jnp.exp(vals[2] - vals[0])
    denom = 1.0 + e1 + e2
    w0 = 1.0 / denom
    w1w = e1 / denom
    w2w = e2 / denom
    rw = (jnp.where(onehots[0], w0, 0.0) + jnp.where(onehots[1], w1w, 0.0)
          + jnp.where(onehots[2], w2w, 0.0))

    fused = rw[:, 0:1] * outs[0]
    for e in range(1, _NUM_EXPERTS):
        fused = fused + rw[:, e:e + 1] * outs[e]
    h = _dot(fused, cw1[:]) + cvec[0:1, :]
    h = _gelu(_ln(h, cvec[1:2, :], cvec[2:3, :]))
    out_ref[:] = _dot(h, cw2[:]) + cb2[:]

    sel = (rw > 0).astype(jnp.float32)
    fsum = jnp.sum(sel, axis=0, keepdims=True)
    mx = jnp.max(logits, axis=1, keepdims=True)
    p = jnp.exp(logits - mx)
    p = p / jnp.sum(p, axis=1, keepdims=True)
    psum = jnp.sum(p, axis=0, keepdims=True)

    @pl.when(i == 0)
    def _():
        freq_acc[:] = fsum
        prob_acc[:] = psum

    @pl.when(i > 0)
    def _():
        freq_acc[:] = freq_acc[:] + fsum
        prob_acc[:] = prob_acc[:] + psum

    @pl.when(i == n_grid - 1)
    def _():
        total = jnp.sum(freq_acc[:] * prob_acc[:])
        scale = 0.01 * float(_NUM_EXPERTS) / (float(_BATCH) * float(_BATCH))
        aux_ref[:] = (scale * total).reshape(1, 1)


@jax.jit
def kernel(x, params):
    inputs = [x]
    in_specs = [pl.BlockSpec((_TILE, _D_IN), lambda i: (i, 0))]

    def add_full(arr):
        shape = arr.shape
        inputs.append(arr)
        in_specs.append(pl.BlockSpec(shape, lambda i: (0,) * len(shape)))

    for (s, e), p in zip(_GROUP_SLICES, params['groups']):
        w1p = jnp.zeros((_D_IN, p['W1'].shape[1]), jnp.float32)
        w1p = w1p.at[s:e, :].set(p['W1'])
        add_full(w1p)
        add_full(jnp.stack([p['b1'], p['g1'], p['bb1']]))
        add_full(p['W2'])
        add_full(jnp.stack([p['b2'], p['g2'], p['bb2']]))
    for p in params['shared']:
        add_full(p['W1'])
        add_full(jnp.stack([p['b1'], p['g1'], p['bb1']]))
        add_full(p['W2'])
        add_full(jnp.stack([p['b2'], p['g2'], p['bb2']]))
    gp = params['gate']
    add_full(gp['W1'])
    add_full(gp['b1'].reshape(1, -1))
    add_full(gp['W2'])
    add_full(gp['b2'].reshape(1, -1))
    cp = params['cls']
    add_full(cp['W1'])
    add_full(jnp.stack([cp['b1'], cp['g'], cp['bb']]))
    add_full(cp['W2'])
    add_full(cp['b2'].reshape(1, -1))

    out_logits, aux = pl.pallas_call(
        _moe_kernel,
        grid=(_BATCH // _TILE,),
        in_specs=in_specs,
        out_specs=[
            pl.BlockSpec((_TILE, 2), lambda i: (i, 0)),
            pl.BlockSpec((1, 1), lambda i: (0, 0)),
        ],
        out_shape=[
            jax.ShapeDtypeStruct((_BATCH, 2), jnp.float32),
            jax.ShapeDtypeStruct((1, 1), jnp.float32),
        ],
        scratch_shapes=[
            pltpu.VMEM((1, _NUM_EXPERTS), jnp.float32),
            pltpu.VMEM((1, _NUM_EXPERTS), jnp.float32),
        ],
    )(*inputs)
    return out_logits, aux[0, 0]
